# baseline (speedup 1.0000x reference)
import functools

import jax
import jax.numpy as jnp
from jax import lax
from jax.experimental import pallas as pl
from jax.experimental.pallas import tpu as pltpu
from jax.experimental.pallas import tpu_sc as plsc

BATCH = 16384
DIM = 32
LANES = 16
NUM_CORES = 2
NUM_SUBCORES = 16
NUM_WORKERS = NUM_CORES * NUM_SUBCORES
B_PER_W = BATCH // NUM_WORKERS
WIN = 512
NFULL = 1000000 // WIN
TAIL_LO = NFULL * WIN
TAIL_N = 1000000 - TAIL_LO
K_PER_W = 62
XROWS = BATCH + 2 * NUM_WORKERS

_MESH = plsc.VectorSubcoreMesh(core_axis_name="c", subcore_axis_name="s")
_CP = pltpu.CompilerParams(needs_layout_passes=False)

def _extract_and_scatter(tab_v, width, win_lo, ids16, pos16, msk, asm_v,
                         x_hbm, dump_row, sem, iota):
    lane = jnp.where(msk, ids16 - win_lo, 0)
    for d in range(DIM):
        dv = jnp.full((LANES,), d, jnp.int32)
        vals = plsc.load_gather(tab_v, [dv, lane])
        plsc.store_scatter(asm_v, [iota, dv], vals, mask=msk)
    rows = jnp.where(msk, pos16, dump_row)
    return pltpu.async_copy(asm_v, x_hbm.at[rows], sem)


def _k1_body_one_table(ids_hbm, tab_hbm, x_hbm, wid, ids_v, lst_id_v,
                       lst_pos_v, whit_id_v, whit_pos_v, winbufs, tail_v,
                       asmbufs, sems, semws, iota):
    pltpu.sync_copy(ids_hbm, ids_v.at[pl.ds(0, BATCH)])
    dump_row = jnp.int32(BATCH + 2 * wid)

    def part(i, cnt):
        v = ids_v[pl.ds(i * LANES, LANES)]
        m = ((v >> 9) & (NUM_WORKERS - 1)) == wid
        plsc.store_compressed(lst_id_v.at[pl.ds(cnt, LANES)], v, mask=m)
        plsc.store_compressed(
            lst_pos_v.at[pl.ds(cnt, LANES)], i * LANES + iota, mask=m)
        return cnt + plsc.all_reduce_population_count(m)[0]

    cnt = lax.fori_loop(0, BATCH // LANES, part, jnp.int32(0))
    nv = (cnt + LANES - 1) // LANES

    def issue_win(win, slot):
        pltpu.async_copy(
            tab_hbm.at[:, pl.ds(pl.multiple_of(win * WIN, 128), WIN)],
            winbufs[slot], semws[slot])

    def process(win, width, tab_ref, win_lo_static):
        win_lo = win_lo_static if width == TAIL_N else win * WIN

        def sel_a(i, wcnt):
            v = lst_id_v[pl.ds(i * LANES, LANES)]
            p = lst_pos_v[pl.ds(i * LANES, LANES)]
            valid = (i * LANES + iota) < cnt
            m = valid & ((v >> 9) == win)
            plsc.store_compressed(whit_id_v.at[pl.ds(wcnt, LANES)], v, mask=m)
            plsc.store_compressed(whit_pos_v.at[pl.ds(wcnt, LANES)], p, mask=m)
            return wcnt + plsc.all_reduce_population_count(m)[0]

        wcnt = lax.fori_loop(0, nv, sel_a, jnp.int32(0))
        tt = (wcnt + 2 * LANES - 1) // (2 * LANES)

        def sel_p(t, carry):
            for slot in (0, 1):
                g = 2 * t + slot

                @pl.when(t > 0)
                def _():
                    pltpu.make_async_copy(
                        asmbufs[slot], x_hbm.at[pl.ds(0, LANES)],
                        sems[slot]).wait()

                ids16 = whit_id_v[pl.ds(g * LANES, LANES)]
                pos16 = whit_pos_v[pl.ds(g * LANES, LANES)]
                m = (g * LANES + iota) < wcnt
                _extract_and_scatter(
                    tab_ref, width, win_lo, ids16, pos16, m, asmbufs[slot],
                    x_hbm, dump_row, sems[slot], iota)
            return carry

        lax.fori_loop(0, tt, sel_p, jnp.int32(0))

        @pl.when(tt > 0)
        def _():
            for slot in (0, 1):
                pltpu.make_async_copy(
                    asmbufs[slot], x_hbm.at[pl.ds(0, LANES)],
                    sems[slot]).wait()

    @pl.when(wid < NFULL)
    def _():
        issue_win(wid, 0)

    @pl.loop(0, K_PER_W // 2)
    def _(kk):
        for slot in (0, 1):
            k = kk * 2 + slot
            win = wid + NUM_WORKERS * k

            @pl.when(win < NFULL)
            def _():
                pltpu.make_async_copy(
                    tab_hbm.at[:, pl.ds(0, WIN)], winbufs[slot],
                    semws[slot]).wait()

                @pl.when(win + NUM_WORKERS < NFULL)
                def _():
                    issue_win(win + NUM_WORKERS, slot ^ 1)

                process(win, WIN, winbufs[slot], 0)

    @pl.when(wid == NFULL % NUM_WORKERS)
    def _():
        pltpu.async_copy(
            tab_hbm.at[:, pl.ds(TAIL_LO, TAIL_N)], tail_v, semws[0]).wait()
        process(jnp.int32(NFULL), TAIL_N, tail_v, TAIL_LO)


@functools.partial(
    pl.kernel,
    out_type=(
        jax.ShapeDtypeStruct((XROWS, 128), jnp.float32),
        jax.ShapeDtypeStruct((XROWS, 128), jnp.float32),
    ),
    mesh=_MESH,
    compiler_params=_CP,
    scratch_types=[
        pltpu.VMEM((BATCH + LANES,), jnp.int32),
        pltpu.VMEM((BATCH + LANES,), jnp.int32),
        pltpu.VMEM((BATCH + LANES,), jnp.int32),
        pltpu.VMEM((BATCH + 2 * LANES,), jnp.int32),
        pltpu.VMEM((BATCH + 2 * LANES,), jnp.int32),
        pltpu.VMEM((DIM, WIN), jnp.float32),
        pltpu.VMEM((DIM, WIN), jnp.float32),
        pltpu.VMEM((DIM, TAIL_N), jnp.float32),
        pltpu.VMEM((LANES, 128), jnp.float32),
        pltpu.VMEM((LANES, 128), jnp.float32),
        pltpu.SemaphoreType.DMA,
        pltpu.SemaphoreType.DMA,
        pltpu.SemaphoreType.DMA,
        pltpu.SemaphoreType.DMA,
    ],
)
def _k1(uids_hbm, iids_hbm, utab_hbm, itab_hbm, xu_hbm, xi_hbm,
        ids_v, lst_id_v, lst_pos_v, whit_id_v, whit_pos_v, win0_v, win1_v,
        tail_v, asm0_v, asm1_v, sem0, sem1, semw0, semw1):
    wid = lax.axis_index("s") * NUM_CORES + lax.axis_index("c")
    iota = lax.iota(jnp.int32, LANES)
    winbufs = (win0_v, win1_v)
    asmbufs = (asm0_v, asm1_v)
    sems = (sem0, sem1)
    semws = (semw0, semw1)
    _k1_body_one_table(uids_hbm, utab_hbm, xu_hbm, wid, ids_v, lst_id_v,
                       lst_pos_v, whit_id_v, whit_pos_v, winbufs, tail_v,
                       asmbufs, sems, semws, iota)
    _k1_body_one_table(iids_hbm, itab_hbm, xi_hbm, wid, ids_v, lst_id_v,
                       lst_pos_v, whit_id_v, whit_pos_v, winbufs, tail_v,
                       asmbufs, sems, semws, iota)


CHUNK = 128


@functools.partial(
    pl.kernel,
    out_type=jax.ShapeDtypeStruct((BATCH,), jnp.float32),
    mesh=_MESH,
    compiler_params=_CP,
    scratch_types=[
        pltpu.VMEM((CHUNK, 128), jnp.float32),
        pltpu.VMEM((CHUNK, 128), jnp.float32),
        pltpu.VMEM((B_PER_W,), jnp.float32),
        pltpu.SemaphoreType.DMA,
        pltpu.SemaphoreType.DMA,
    ],
)
def _k2(xu_hbm, xi_hbm, out_hbm, u_v, i_v, out_v, semu, semi):
    wid = lax.axis_index("s") * NUM_CORES + lax.axis_index("c")
    base = wid * B_PER_W
    iota = lax.iota(jnp.int32, LANES)

    @pl.loop(0, B_PER_W // CHUNK)
    def _(cc):
        r0 = base + cc * CHUNK
        cu = pltpu.async_copy(xu_hbm.at[pl.ds(r0, CHUNK)], u_v, semu)
        ci = pltpu.async_copy(xi_hbm.at[pl.ds(r0, CHUNK)], i_v, semi)
        cu.wait()
        ci.wait()

        @pl.loop(0, CHUNK // LANES)
        def _(g):
            rows = g * LANES + iota
            acc = jnp.zeros((LANES,), jnp.float32)
            for d in range(DIM):
                dv = jnp.full((LANES,), d, jnp.int32)
                acc = acc + (plsc.load_gather(u_v, [rows, dv])
                             * plsc.load_gather(i_v, [rows, dv]))
            out_v[pl.ds(cc * CHUNK + g * LANES, LANES)] = acc

    pltpu.sync_copy(out_v, out_hbm.at[pl.ds(base, B_PER_W)])


def kernel(user_ids, item_ids, user_table, item_table):
    user_ids = user_ids.astype(jnp.int32)
    item_ids = item_ids.astype(jnp.int32)
    xu, xi = _k1(user_ids, item_ids, user_table.T, item_table.T)
    return _k2(xu, xi)


# speedup vs baseline: 1.8130x; 1.8130x over previous
import functools

import jax
import jax.numpy as jnp
from jax import lax
from jax.experimental import pallas as pl
from jax.experimental.pallas import tpu as pltpu
from jax.experimental.pallas import tpu_sc as plsc

BATCH = 16384
DIM = 32
LANES = 16
NUM_CORES = 2
NUM_SUBCORES = 16
NUM_WORKERS = NUM_CORES * NUM_SUBCORES
B_PER_W = BATCH // NUM_WORKERS
DEPTH = 8

GROUP_ITERS = LANES // DEPTH

_MESH = plsc.VectorSubcoreMesh(core_axis_name="c", subcore_axis_name="s")
_CP = pltpu.CompilerParams(needs_layout_passes=False)


@functools.partial(
    pl.kernel,
    out_type=jax.ShapeDtypeStruct((BATCH,), jnp.float32),
    mesh=_MESH,
    compiler_params=_CP,
    scratch_types=[
        pltpu.VMEM((DEPTH, DIM, 128), jnp.float32),
        pltpu.VMEM((DEPTH, DIM, 128), jnp.float32),
        pltpu.VMEM((LANES, LANES), jnp.float32),
        pltpu.VMEM((B_PER_W,), jnp.float32),
        pltpu.VMEM((B_PER_W + LANES,), jnp.int32),
        pltpu.VMEM((B_PER_W + LANES,), jnp.int32),
    ] + [pltpu.SemaphoreType.DMA] * (2 * DEPTH),
)
def _two_tower_sc(uids_hbm, iids_hbm, utabT_hbm, itabT_hbm, out_hbm,
                  ublk_v, iblk_v, pbuf_v, out_v,
                  uids_v, iids_v, *sems):
    usem = sems[:DEPTH]
    isem = sems[DEPTH:]
    wid = lax.axis_index("s") * NUM_CORES + lax.axis_index("c")
    base = wid * B_PER_W

    pltpu.sync_copy(uids_hbm.at[pl.ds(base, B_PER_W)], uids_v.at[pl.ds(0, B_PER_W)])
    pltpu.sync_copy(iids_hbm.at[pl.ds(base, B_PER_W)], iids_v.at[pl.ds(0, B_PER_W)])

    def issue(j, r):
        uid = uids_v[pl.ds(j, LANES)][0]
        iid = iids_v[pl.ds(j, LANES)][0]
        ub = pl.multiple_of(uid & ~127, 128)
        ib = pl.multiple_of(iid & ~127, 128)
        pltpu.async_copy(utabT_hbm.at[:, pl.ds(ub, 128)], ublk_v.at[r], usem[r])
        pltpu.async_copy(itabT_hbm.at[:, pl.ds(ib, 128)], iblk_v.at[r], isem[r])

    for r in range(DEPTH):
        issue(r, r)

    iota = lax.iota(jnp.int32, LANES)
    niters = B_PER_W // DEPTH

    @pl.loop(0, niters)
    def _(jj):
        for r in range(DEPTH):
            j = jj * DEPTH + r
            pltpu.make_async_copy(
                utabT_hbm.at[:, pl.ds(0, 128)], ublk_v.at[r], usem[r]).wait()
            pltpu.make_async_copy(
                itabT_hbm.at[:, pl.ds(0, 128)], iblk_v.at[r], isem[r]).wait()

            ul = jnp.full((LANES,), uids_v[pl.ds(j, LANES)][0] & 127, jnp.int32)
            il = jnp.full((LANES,), iids_v[pl.ds(j, LANES)][0] & 127, jnp.int32)
            rr = jnp.full((LANES,), r, jnp.int32)
            u0 = plsc.load_gather(ublk_v, [rr, iota, ul])
            u1 = plsc.load_gather(ublk_v, [rr, iota + LANES, ul])
            i0 = plsc.load_gather(iblk_v, [rr, iota, il])
            i1 = plsc.load_gather(iblk_v, [rr, iota + LANES, il])
            p = u0 * i0 + u1 * i1

            row = (jj % GROUP_ITERS) * DEPTH + r
            pbuf_v[row] = p

            @pl.when(jj < niters - 1)
            def _():
                issue(j + DEPTH, r)

            if r == DEPTH - 1:
                @pl.when(jj % GROUP_ITERS == GROUP_ITERS - 1)
                def _():
                    acc = jnp.zeros((LANES,), jnp.float32)
                    for d in range(LANES):
                        acc = acc + plsc.load_gather(
                            pbuf_v, [iota, jnp.full((LANES,), d, jnp.int32)])
                    g0 = (jj - (GROUP_ITERS - 1)) * DEPTH
                    out_v[pl.ds(g0, LANES)] = acc

    pltpu.sync_copy(out_v, out_hbm.at[pl.ds(base, B_PER_W)])


def kernel(user_ids, item_ids, user_table, item_table):
    user_ids = user_ids.astype(jnp.int32)
    item_ids = item_ids.astype(jnp.int32)
    return _two_tower_sc(user_ids, item_ids, user_table.T, item_table.T)
